# baseline (speedup 1.0000x reference)
import jax
import jax.numpy as jnp
from jax.experimental import pallas as pl
from jax.experimental.pallas import tpu as pltpu

N = 4096
NFEAT = 512
NHID = 128
NHEADS = 4
NOUT = 128
ALPHA = 0.2

BR = 256


def _stats_kernel(feat_ref, wf_ref, cf_ref, rowv_ref, bv_ref):
    f = (
        jnp.dot(feat_ref[...], wf_ref[...], preferred_element_type=jnp.float32)
        + cf_ref[...]
    )
    f1 = f[:, 0:NHEADS]
    f2 = f[:, NHEADS : 2 * NHEADS]
    m2 = jnp.max(f2, axis=0, keepdims=True)
    t = f1 + m2
    m = jnp.maximum(t, ALPHA * t)
    rowv_ref[:, 0:NHEADS] = jnp.exp(t - m)
    rowv_ref[:, NHEADS : 2 * NHEADS] = jnp.exp(ALPHA * t - m)
    u = f2 - m2
    bv_ref[:, 0:NHEADS] = jnp.exp(u)
    bv_ref[:, NHEADS : 2 * NHEADS] = jnp.exp(ALPHA * u)


def _stats(feat, wf, cf):
    return pl.pallas_call(
        _stats_kernel,
        out_shape=[
            jax.ShapeDtypeStruct((N, 2 * NHEADS), jnp.float32),
            jax.ShapeDtypeStruct((N, 2 * NHEADS), jnp.float32),
        ],
    )(feat, wf, cf)


CHUNK = 512


def _heads(adj_ref, rowv_ref, colv_ref, v_scr):
    us = [jnp.zeros((BR, NHID), jnp.float32) for _ in range(NHEADS)]
    dens = [jnp.zeros((BR, 1), jnp.float32) for _ in range(NHEADS)]
    for k in range(N // CHUNK):
        adj = adj_ref[:, k * CHUNK : (k + 1) * CHUNK]
        for i in range(NHEADS):
            ap = rowv_ref[:, i : i + 1]
            an = rowv_ref[:, NHEADS + i : NHEADS + i + 1]
            bp = colv_ref[i : i + 1, k * CHUNK : (k + 1) * CHUNK]
            bn = colv_ref[NHEADS + i : NHEADS + i + 1,
                          k * CHUNK : (k + 1) * CHUNK]
            p = jnp.maximum(ap * bp, an * bn) * adj
            us[i] += jnp.dot(
                p.astype(jnp.bfloat16),
                v_scr[k * CHUNK : (k + 1) * CHUNK, i * NHID : (i + 1) * NHID],
                preferred_element_type=jnp.float32,
            )
            dens[i] += jnp.sum(p, axis=1, keepdims=True)
    return us, dens


def _norm_elu(us, dens):
    cols = []
    for i in range(NHEADS):
        d = dens[i]
        ok = d > 0.0
        x = us[i] / jnp.where(ok, d, 1.0)
        x = jnp.where(ok, x, 0.0)
        cols.append(jnp.where(x > 0.0, x, jnp.exp(x) - 1.0))
    return jnp.concatenate(cols, axis=1)


def _mm_kernel(x_ref, w_ref, b_ref, o_ref):
    o_ref[...] = (
        jnp.dot(x_ref[...], w_ref[...], preferred_element_type=jnp.float32)
        + b_ref[...]
    ).astype(jnp.bfloat16)


def _mm(x, w, b):
    k = x.shape[1]
    m = w.shape[1]
    return pl.pallas_call(
        _mm_kernel,
        grid=(N // BR,),
        in_specs=[
            pl.BlockSpec((BR, k), lambda r: (r, 0)),
            pl.BlockSpec((k, m), lambda r: (0, 0)),
            pl.BlockSpec((1, m), lambda r: (0, 0)),
        ],
        out_specs=pl.BlockSpec((BR, m), lambda r: (r, 0)),
        out_shape=jax.ShapeDtypeStruct((N, m), jnp.bfloat16),
        compiler_params=pltpu.CompilerParams(
            dimension_semantics=("parallel",),
        ),
    )(x, w, b)


def _att1_kernel(adj_ref, rowv_ref, colv_ref, v_ref, out_ref, den_ref):
    us, dens = _heads(adj_ref[...], rowv_ref, colv_ref, v_ref)
    den_ref[...] = jnp.concatenate(dens, axis=1)
    out_ref[...] = _norm_elu(us, dens).astype(jnp.bfloat16)


def _att1_pass(adjs, rowv, colv, v0):
    grid = (N // BR,)
    return pl.pallas_call(
        _att1_kernel,
        grid=grid,
        in_specs=[
            pl.BlockSpec((BR, N), lambda r: (r, 0)),
            pl.BlockSpec((BR, 2 * NHEADS), lambda r: (r, 0)),
            pl.BlockSpec((2 * NHEADS, N), lambda r: (0, 0)),
            pl.BlockSpec((N, NHEADS * NHID), lambda r: (0, 0)),
        ],
        out_specs=[
            pl.BlockSpec((BR, NHEADS * NHID), lambda r: (r, 0)),
            pl.BlockSpec((BR, NHEADS), lambda r: (r, 0)),
        ],
        out_shape=[
            jax.ShapeDtypeStruct((N, NHEADS * NHID), jnp.bfloat16),
            jax.ShapeDtypeStruct((N, NHEADS), jnp.float32),
        ],
        compiler_params=pltpu.CompilerParams(
            dimension_semantics=("parallel",),
        ),
    )(adjs, rowv, colv, v0)


def _att2_kernel(adj_ref, rowv_ref, colv_ref, v_ref, den_ref,
                 wl_ref, bl_ref, out_ref):
    us, _ = _heads(adj_ref[...], rowv_ref, colv_ref, v_ref)
    den = den_ref[...]
    dens = [den[:, i : i + 1] for i in range(NHEADS)]
    x = _norm_elu(us, dens)
    out_ref[...] = (
        jnp.dot(
            x.astype(jnp.bfloat16), wl_ref[...],
            preferred_element_type=jnp.float32,
        )
        + bl_ref[...]
    )


def _att2_pass(adjs, rowv, colv, v1, den, wl, bl):
    grid = (N // BR,)
    return pl.pallas_call(
        _att2_kernel,
        grid=grid,
        in_specs=[
            pl.BlockSpec((BR, N), lambda r: (r, 0)),
            pl.BlockSpec((BR, 2 * NHEADS), lambda r: (r, 0)),
            pl.BlockSpec((2 * NHEADS, N), lambda r: (0, 0)),
            pl.BlockSpec((N, NHEADS * NHID), lambda r: (0, 0)),
            pl.BlockSpec((BR, NHEADS), lambda r: (r, 0)),
            pl.BlockSpec((NHEADS * NHID, NOUT), lambda r: (0, 0)),
            pl.BlockSpec((1, NOUT), lambda r: (0, 0)),
        ],
        out_specs=pl.BlockSpec((BR, NOUT), lambda r: (r, 0)),
        out_shape=jax.ShapeDtypeStruct((N, NOUT), jnp.float32),
        compiler_params=pltpu.CompilerParams(
            dimension_semantics=("parallel",),
        ),
    )(adjs, rowv, colv, v1, den, wl, bl)


@jax.jit
def kernel(feat_data, adjs, fW_W, fW_b, a_src, a_dest, W0, b0, W1, b1, Wl, bl):
    w_src = jnp.einsum("hfk,hk->fh", fW_W, a_src)
    w_dst = jnp.einsum("hfk,hk->fh", fW_W, a_dest)
    wf = jnp.concatenate([w_src, w_dst], axis=1)
    cf = jnp.concatenate(
        [jnp.sum(fW_b * a_src, axis=1), jnp.sum(fW_b * a_dest, axis=1)]
    )[None, :]
    w0cat = jnp.concatenate(list(W0), axis=1).astype(jnp.bfloat16)
    b0cat = jnp.concatenate(list(b0))[None, :]
    w1cat = jnp.concatenate(list(W1), axis=1).astype(jnp.bfloat16)
    b1cat = jnp.concatenate(list(b1))[None, :]
    feat_bf = feat_data.astype(jnp.bfloat16)

    rowv, bv = _stats(feat_data, wf, cf)
    colv = bv.T

    v0 = _mm(feat_bf, w0cat, b0cat)
    x1, den = _att1_pass(adjs, rowv, colv, v0)
    v1 = _mm(x1, w1cat, b1cat)
    out = _att2_pass(adjs, rowv, colv, v1, den,
                     Wl.astype(jnp.bfloat16), bl[None, :])
    return out


# speedup vs baseline: 1.3153x; 1.3153x over previous
import jax
import jax.numpy as jnp
from jax.experimental import pallas as pl
from jax.experimental.pallas import tpu as pltpu

N = 4096
NFEAT = 512
NHID = 128
NHEADS = 4
NOUT = 128
ALPHA = 0.2

BR = 256


def _stats_kernel(feat_ref, wf_ref, cf_ref, rowv_ref, bv_ref):
    f = (
        jnp.dot(feat_ref[...], wf_ref[...], preferred_element_type=jnp.float32)
        + cf_ref[...]
    )
    f1 = f[:, 0:NHEADS]
    f2 = f[:, NHEADS : 2 * NHEADS]
    m2 = jnp.max(f2, axis=0, keepdims=True)
    t = f1 + m2
    m = jnp.maximum(t, ALPHA * t)
    rowv_ref[:, 0:NHEADS] = jnp.exp(t - m)
    rowv_ref[:, NHEADS : 2 * NHEADS] = jnp.exp(ALPHA * t - m)
    u = f2 - m2
    bv_ref[:, 0:NHEADS] = jnp.exp(u)
    bv_ref[:, NHEADS : 2 * NHEADS] = jnp.exp(ALPHA * u)


def _stats(feat, wf, cf):
    return pl.pallas_call(
        _stats_kernel,
        out_shape=[
            jax.ShapeDtypeStruct((N, 2 * NHEADS), jnp.float32),
            jax.ShapeDtypeStruct((N, 2 * NHEADS), jnp.float32),
        ],
    )(feat, wf, cf)


CHUNK = 512


VW = 2 * NHID


def _heads(adj_ref, rowv_ref, colv_ref, v_scr):
    uf = [jnp.zeros((BR, VW), jnp.float32) for _ in range(NHEADS)]
    for k in range(N // CHUNK):
        adj = adj_ref[:, k * CHUNK : (k + 1) * CHUNK]
        for i in range(NHEADS):
            ap = rowv_ref[:, i : i + 1]
            an = rowv_ref[:, NHEADS + i : NHEADS + i + 1]
            bp = colv_ref[i : i + 1, k * CHUNK : (k + 1) * CHUNK]
            bn = colv_ref[NHEADS + i : NHEADS + i + 1,
                          k * CHUNK : (k + 1) * CHUNK]
            p = jnp.maximum(ap * bp, an * bn) * adj
            uf[i] += jnp.dot(
                p.astype(jnp.bfloat16),
                v_scr[k * CHUNK : (k + 1) * CHUNK, i * VW : (i + 1) * VW],
                preferred_element_type=jnp.float32,
            )
    us = [u[:, :NHID] for u in uf]
    dens = [u[:, NHID : NHID + 1] for u in uf]
    return us, dens


def _norm_elu(us, dens):
    cols = []
    for i in range(NHEADS):
        d = dens[i]
        ok = d > 0.0
        x = us[i] / jnp.where(ok, d, 1.0)
        x = jnp.where(ok, x, 0.0)
        cols.append(jnp.where(x > 0.0, x, jnp.exp(x) - 1.0))
    return jnp.concatenate(cols, axis=1)


def _att1_kernel(adj_ref, rowv_ref, colv_ref, x_ref, w_ref, b_ref,
                 out_ref, v_scr):
    r = pl.program_id(0)

    @pl.when(r == 0)
    def _make_v():
        vv = (
            jnp.dot(x_ref[...], w_ref[...], preferred_element_type=jnp.float32)
            + b_ref[...]
        )
        pad = jnp.concatenate(
            [jnp.ones((N, 1), jnp.float32), jnp.zeros((N, NHID - 1), jnp.float32)],
            axis=1,
        ).astype(jnp.bfloat16)
        for i in range(NHEADS):
            v_scr[:, i * VW : i * VW + NHID] = (
                vv[:, i * NHID : (i + 1) * NHID].astype(jnp.bfloat16)
            )
            v_scr[:, i * VW + NHID : (i + 1) * VW] = pad

    us, dens = _heads(adj_ref[...], rowv_ref, colv_ref, v_scr)
    out_ref[...] = _norm_elu(us, dens).astype(jnp.bfloat16)


def _att1_pass(adjs, rowv, colv, feat_bf, w0cat, b0cat):
    grid = (N // BR,)
    return pl.pallas_call(
        _att1_kernel,
        grid=grid,
        in_specs=[
            pl.BlockSpec((BR, N), lambda r: (r, 0)),
            pl.BlockSpec((BR, 2 * NHEADS), lambda r: (r, 0)),
            pl.BlockSpec((2 * NHEADS, N), lambda r: (0, 0)),
            pl.BlockSpec((N, NFEAT), lambda r: (0, 0)),
            pl.BlockSpec((NFEAT, NHEADS * NHID), lambda r: (0, 0)),
            pl.BlockSpec((1, NHEADS * NHID), lambda r: (0, 0)),
        ],
        out_specs=pl.BlockSpec((BR, NHEADS * NHID), lambda r: (r, 0)),
        out_shape=jax.ShapeDtypeStruct((N, NHEADS * NHID), jnp.bfloat16),
        scratch_shapes=[
            pltpu.VMEM((N, NHEADS * VW), jnp.bfloat16),
        ],
        compiler_params=pltpu.CompilerParams(
            dimension_semantics=("arbitrary",),
        ),
    )(adjs, rowv, colv, feat_bf, w0cat, b0cat)


def _att2_kernel(adj_ref, rowv_ref, colv_ref, x_ref, w_ref, b_ref,
                 wl_ref, bl_ref, out_ref, v_scr):
    r = pl.program_id(0)

    @pl.when(r == 0)
    def _make_v():
        vv = (
            jnp.dot(x_ref[...], w_ref[...], preferred_element_type=jnp.float32)
            + b_ref[...]
        )
        pad = jnp.concatenate(
            [jnp.ones((N, 1), jnp.float32), jnp.zeros((N, NHID - 1), jnp.float32)],
            axis=1,
        ).astype(jnp.bfloat16)
        for i in range(NHEADS):
            v_scr[:, i * VW : i * VW + NHID] = (
                vv[:, i * NHID : (i + 1) * NHID].astype(jnp.bfloat16)
            )
            v_scr[:, i * VW + NHID : (i + 1) * VW] = pad

    us, dens = _heads(adj_ref[...], rowv_ref, colv_ref, v_scr)
    x = _norm_elu(us, dens)
    out_ref[...] = (
        jnp.dot(
            x.astype(jnp.bfloat16), wl_ref[...],
            preferred_element_type=jnp.float32,
        )
        + bl_ref[...]
    )


def _att2_pass(adjs, rowv, colv, x1, w1cat, b1cat, wl, bl):
    grid = (N // BR,)
    return pl.pallas_call(
        _att2_kernel,
        grid=grid,
        in_specs=[
            pl.BlockSpec((BR, N), lambda r: (r, 0)),
            pl.BlockSpec((BR, 2 * NHEADS), lambda r: (r, 0)),
            pl.BlockSpec((2 * NHEADS, N), lambda r: (0, 0)),
            pl.BlockSpec((N, NHEADS * NHID), lambda r: (0, 0)),
            pl.BlockSpec((NHEADS * NHID, NHEADS * NHID), lambda r: (0, 0)),
            pl.BlockSpec((1, NHEADS * NHID), lambda r: (0, 0)),
            pl.BlockSpec((NHEADS * NHID, NOUT), lambda r: (0, 0)),
            pl.BlockSpec((1, NOUT), lambda r: (0, 0)),
        ],
        out_specs=pl.BlockSpec((BR, NOUT), lambda r: (r, 0)),
        out_shape=jax.ShapeDtypeStruct((N, NOUT), jnp.float32),
        scratch_shapes=[
            pltpu.VMEM((N, NHEADS * VW), jnp.bfloat16),
        ],
        compiler_params=pltpu.CompilerParams(
            dimension_semantics=("arbitrary",),
        ),
    )(adjs, rowv, colv, x1, w1cat, b1cat, wl, bl)


@jax.jit
def kernel(feat_data, adjs, fW_W, fW_b, a_src, a_dest, W0, b0, W1, b1, Wl, bl):
    w_src = jnp.einsum("hfk,hk->fh", fW_W, a_src)
    w_dst = jnp.einsum("hfk,hk->fh", fW_W, a_dest)
    wf = jnp.concatenate([w_src, w_dst], axis=1)
    cf = jnp.concatenate(
        [jnp.sum(fW_b * a_src, axis=1), jnp.sum(fW_b * a_dest, axis=1)]
    )[None, :]
    w0cat = jnp.concatenate(list(W0), axis=1).astype(jnp.bfloat16)
    b0cat = jnp.concatenate(list(b0))[None, :]
    w1cat = jnp.concatenate(list(W1), axis=1).astype(jnp.bfloat16)
    b1cat = jnp.concatenate(list(b1))[None, :]
    feat_bf = feat_data.astype(jnp.bfloat16)

    rowv, bv = _stats(feat_data, wf, cf)
    colv = bv.T

    x1 = _att1_pass(adjs, rowv, colv, feat_bf, w0cat, b0cat)
    out = _att2_pass(adjs, rowv, colv, x1, w1cat, b1cat,
                     Wl.astype(jnp.bfloat16), bl[None, :])
    return out
